# baseline (speedup 1.0000x reference)
import functools
import math

import jax
import jax.numpy as jnp
from jax import lax
from jax.experimental import pallas as pl
from jax.experimental.pallas import tpu as pltpu
from jax.experimental.pallas import tpu_sc as plsc

_B = 4096
_F = 26
_D = 128
_V = 26000

_NC = 2
_NS = 16
_NW = _NC * _NS
_CHUNK = 128

_F_HALF = _F // 2

_MLP_BB = 128
_MLP_GRID = _B // _MLP_BB

_INV_SQRT2 = 1.0 / math.sqrt(2.0)


@functools.lru_cache(maxsize=None)
def _sc_gather_kernel(n_chunks):
    rpw = n_chunks * _CHUNK
    rows = rpw * _NW

    @functools.partial(
        pl.kernel,
        out_type=jax.ShapeDtypeStruct((rows, _D), jnp.float32),
        mesh=plsc.VectorSubcoreMesh(core_axis_name="c", subcore_axis_name="s"),
        scratch_types=[
            pltpu.VMEM((n_chunks, _CHUNK), jnp.int32),
            pltpu.VMEM((2, _CHUNK, _D), jnp.float32),
            pltpu.SemaphoreType.DMA((2,)),
            pltpu.SemaphoreType.DMA((2,)),
        ],
    )
    def _sc_gather(table_hbm, idx_hbm, out_hbm, idx_v, bufs, gsems, ssems):
        wid = lax.axis_index("s") * _NC + lax.axis_index("c")
        base = wid * rpw
        pltpu.sync_copy(idx_hbm.at[wid], idx_v)

        for b in range(2):
            pltpu.async_copy(table_hbm.at[idx_v.at[b]], bufs.at[b], gsems.at[b])

        def body(j, carry):
            b = lax.rem(j, 2)
            pltpu.make_async_copy(
                table_hbm.at[pl.ds(0, _CHUNK)], bufs.at[b], gsems.at[b]
            ).wait()
            pltpu.async_copy(
                bufs.at[b], out_hbm.at[pl.ds(base + j * _CHUNK, _CHUNK)], ssems.at[b]
            )

            @pl.when(j + 2 < n_chunks)
            def _():
                pltpu.make_async_copy(
                    table_hbm.at[pl.ds(0, _CHUNK)], bufs.at[b], ssems.at[b]
                ).wait()
                pltpu.async_copy(
                    table_hbm.at[idx_v.at[j + 2]], bufs.at[b], gsems.at[b]
                )

            return carry

        lax.fori_loop(0, n_chunks, body, 0)

        for b in range(2):
            pltpu.make_async_copy(
                table_hbm.at[pl.ds(0, _CHUNK)], bufs.at[b], ssems.at[b]
            ).wait()

    return _sc_gather


@functools.lru_cache(maxsize=None)
def _mlp_call(fc, f_blk, aliased):
    def body(*refs):
        if aliased:
            x_ref, w1_ref, b1_ref, w2_ref, b2_ref, _, o_ref = refs
        else:
            x_ref, w1_ref, b1_ref, w2_ref, b2_ref, o_ref = refs
        x = x_ref[...].reshape(fc * _MLP_BB, _D)
        h = jnp.dot(x, w1_ref[...], preferred_element_type=jnp.float32) + b1_ref[...]
        h = 0.5 * h * (1.0 + lax.erf(h * _INV_SQRT2))
        out = jnp.dot(h, w2_ref[...], preferred_element_type=jnp.float32) + b2_ref[...]
        o_ref[...] = out.reshape(fc, _MLP_BB, _D)

    in_specs = [
        pl.BlockSpec((fc, _MLP_BB, _D), lambda i: (0, i, 0)),
        pl.BlockSpec((_D, _D), lambda i: (0, 0)),
        pl.BlockSpec((1, _D), lambda i: (0, 0)),
        pl.BlockSpec((_D, _D), lambda i: (0, 0)),
        pl.BlockSpec((1, _D), lambda i: (0, 0)),
    ]
    if aliased:
        in_specs.append(pl.BlockSpec(memory_space=pl.ANY))
    return pl.pallas_call(
        body,
        grid=(_MLP_GRID,),
        in_specs=in_specs,
        out_specs=pl.BlockSpec((fc, _MLP_BB, _D), lambda i: (f_blk, i, 0)),
        out_shape=jax.ShapeDtypeStruct((_F, _B, _D), jnp.float32),
        input_output_aliases={5: 0} if aliased else {},
    )


def kernel(attrs, attr_emb, W1, b1, W2, b2):
    shift = (jnp.arange(_F, dtype=attrs.dtype) * 1000)[:, None]
    idx = attrs.T + shift
    b1r, b2r = b1[None, :], b2[None, :]

    half_chunks = _F_HALF * (_B // _CHUNK) // _NW
    gather = _sc_gather_kernel(half_chunks)

    idx0 = idx[:_F_HALF].reshape(_NW, half_chunks, _CHUNK)
    idx1 = idx[_F_HALF:].reshape(_NW, half_chunks, _CHUNK)
    emb0 = gather(attr_emb, idx0).reshape(_F_HALF, _B, _D)
    emb1 = gather(attr_emb, idx1).reshape(_F_HALF, _B, _D)

    out0 = _mlp_call(_F_HALF, 0, False)(emb0, W1, b1r, W2, b2r)
    out3 = _mlp_call(_F_HALF, 1, True)(emb1, W1, b1r, W2, b2r, out0)
    return out3.transpose(1, 0, 2)


# speedup vs baseline: 1.6258x; 1.6258x over previous
import functools
import math

import jax
import jax.numpy as jnp
from jax import lax
from jax.experimental import pallas as pl
from jax.experimental.pallas import tpu as pltpu
from jax.experimental.pallas import tpu_sc as plsc

_B = 4096
_F = 26
_D = 128
_V = 26000
_ROWS = _B * _F

_NC = 2
_NS = 16
_NW = _NC * _NS
_RPW = _ROWS // _NW
_CHUNK = 128
_NCHUNKS = _RPW // _CHUNK

_MLP_ROWS = 2000
_MLP_GRID = _V // _MLP_ROWS

_INV_SQRT2 = 1.0 / math.sqrt(2.0)


def _mlp_body(x_ref, w1_ref, b1_ref, w2_ref, b2_ref, o_ref):
    x = x_ref[...]
    h = jnp.dot(x, w1_ref[...], preferred_element_type=jnp.float32) + b1_ref[...]
    h = 0.5 * h * (1.0 + lax.erf(h * _INV_SQRT2))
    o_ref[...] = jnp.dot(h, w2_ref[...], preferred_element_type=jnp.float32) + b2_ref[...]


def _mlp_table(attr_emb, W1, b1, W2, b2):
    return pl.pallas_call(
        _mlp_body,
        grid=(_MLP_GRID,),
        in_specs=[
            pl.BlockSpec((_MLP_ROWS, _D), lambda i: (i, 0)),
            pl.BlockSpec((_D, _D), lambda i: (0, 0)),
            pl.BlockSpec((1, _D), lambda i: (0, 0)),
            pl.BlockSpec((_D, _D), lambda i: (0, 0)),
            pl.BlockSpec((1, _D), lambda i: (0, 0)),
        ],
        out_specs=pl.BlockSpec((_MLP_ROWS, _D), lambda i: (i, 0)),
        out_shape=jax.ShapeDtypeStruct((_V, _D), jnp.float32),
    )(attr_emb, W1, b1[None, :], W2, b2[None, :])


@functools.lru_cache(maxsize=1)
def _sc_gather_kernel():
    @functools.partial(
        pl.kernel,
        out_type=jax.ShapeDtypeStruct((_ROWS, _D), jnp.float32),
        mesh=plsc.VectorSubcoreMesh(core_axis_name="c", subcore_axis_name="s"),
        scratch_types=[
            pltpu.VMEM((_NCHUNKS, _CHUNK), jnp.int32),
            pltpu.VMEM((2, _CHUNK, _D), jnp.float32),
            pltpu.SemaphoreType.DMA((2,)),
            pltpu.SemaphoreType.DMA((2,)),
        ],
    )
    def _sc_gather(table_hbm, idx_hbm, out_hbm, idx_v, bufs, gsems, ssems):
        wid = lax.axis_index("s") * _NC + lax.axis_index("c")
        base = wid * _RPW
        pltpu.sync_copy(idx_hbm.at[wid], idx_v)

        for b in range(2):
            pltpu.async_copy(table_hbm.at[idx_v.at[b]], bufs.at[b], gsems.at[b])

        def body(j, carry):
            b = lax.rem(j, 2)
            pltpu.make_async_copy(
                table_hbm.at[pl.ds(0, _CHUNK)], bufs.at[b], gsems.at[b]
            ).wait()
            pltpu.async_copy(
                bufs.at[b], out_hbm.at[pl.ds(base + j * _CHUNK, _CHUNK)], ssems.at[b]
            )

            @pl.when(j + 2 < _NCHUNKS)
            def _():
                pltpu.make_async_copy(
                    table_hbm.at[pl.ds(0, _CHUNK)], bufs.at[b], ssems.at[b]
                ).wait()
                pltpu.async_copy(
                    table_hbm.at[idx_v.at[j + 2]], bufs.at[b], gsems.at[b]
                )

            return carry

        lax.fori_loop(0, _NCHUNKS, body, 0)

        for b in range(2):
            pltpu.make_async_copy(
                table_hbm.at[pl.ds(0, _CHUNK)], bufs.at[b], ssems.at[b]
            ).wait()

    return _sc_gather


def kernel(attrs, attr_emb, W1, b1, W2, b2):
    shift = (jnp.arange(_F, dtype=attrs.dtype) * 1000)[:, None]
    idx = (attrs.T + shift).reshape(_NW, _NCHUNKS, _CHUNK)
    out_table = _mlp_table(attr_emb, W1, b1, W2, b2)
    out_flat = _sc_gather_kernel()(out_table, idx)
    return out_flat.reshape(_F, _B, _D).transpose(1, 0, 2)


# speedup vs baseline: 1.6555x; 1.0182x over previous
import functools
import math

import jax
import jax.numpy as jnp
from jax import lax
from jax.experimental import pallas as pl
from jax.experimental.pallas import tpu as pltpu
from jax.experimental.pallas import tpu_sc as plsc

_B = 4096
_F = 26
_D = 128
_V = 26000
_ROWS = _B * _F

_NC = 2
_NS = 16
_NW = _NC * _NS
_RPW = _ROWS // _NW
_CHUNK = 128
_NCHUNKS = _RPW // _CHUNK
_NBUF = 4

_MLP_ROWS = 2000
_MLP_GRID = _V // _MLP_ROWS

_INV_SQRT2 = 1.0 / math.sqrt(2.0)


def _mlp_body(x_ref, w1_ref, b1_ref, w2_ref, b2_ref, o_ref):
    x = x_ref[...]
    h = jnp.dot(x, w1_ref[...], preferred_element_type=jnp.float32) + b1_ref[...]
    h = 0.5 * h * (1.0 + lax.erf(h * _INV_SQRT2))
    o_ref[...] = jnp.dot(h, w2_ref[...], preferred_element_type=jnp.float32) + b2_ref[...]


def _mlp_table(attr_emb, W1, b1, W2, b2):
    return pl.pallas_call(
        _mlp_body,
        grid=(_MLP_GRID,),
        in_specs=[
            pl.BlockSpec((_MLP_ROWS, _D), lambda i: (i, 0)),
            pl.BlockSpec((_D, _D), lambda i: (0, 0)),
            pl.BlockSpec((1, _D), lambda i: (0, 0)),
            pl.BlockSpec((_D, _D), lambda i: (0, 0)),
            pl.BlockSpec((1, _D), lambda i: (0, 0)),
        ],
        out_specs=pl.BlockSpec((_MLP_ROWS, _D), lambda i: (i, 0)),
        out_shape=jax.ShapeDtypeStruct((_V, _D), jnp.float32),
    )(attr_emb, W1, b1[None, :], W2, b2[None, :])


@functools.lru_cache(maxsize=1)
def _sc_gather_kernel():
    @functools.partial(
        pl.kernel,
        out_type=jax.ShapeDtypeStruct((_ROWS, _D), jnp.float32),
        mesh=plsc.VectorSubcoreMesh(core_axis_name="c", subcore_axis_name="s"),
        scratch_types=[
            pltpu.VMEM((_NCHUNKS, _CHUNK), jnp.int32),
            pltpu.VMEM((_NBUF, _CHUNK, _D), jnp.float32),
            pltpu.SemaphoreType.DMA((_NBUF,)),
            pltpu.SemaphoreType.DMA((_NBUF,)),
        ],
    )
    def _sc_gather(table_hbm, idx_hbm, out_hbm, idx_v, bufs, gsems, ssems):
        wid = lax.axis_index("s") * _NC + lax.axis_index("c")
        base = wid * _RPW
        pltpu.sync_copy(idx_hbm.at[wid], idx_v)

        for b in range(_NBUF):
            pltpu.async_copy(table_hbm.at[idx_v.at[b]], bufs.at[b], gsems.at[b])

        def body(j, carry):
            b = lax.rem(j, _NBUF)
            pltpu.make_async_copy(
                table_hbm.at[pl.ds(0, _CHUNK)], bufs.at[b], gsems.at[b]
            ).wait()
            pltpu.async_copy(
                bufs.at[b], out_hbm.at[pl.ds(base + j * _CHUNK, _CHUNK)], ssems.at[b]
            )

            @pl.when(j + _NBUF < _NCHUNKS)
            def _():
                pltpu.make_async_copy(
                    table_hbm.at[pl.ds(0, _CHUNK)], bufs.at[b], ssems.at[b]
                ).wait()
                pltpu.async_copy(
                    table_hbm.at[idx_v.at[j + _NBUF]], bufs.at[b], gsems.at[b]
                )

            return carry

        lax.fori_loop(0, _NCHUNKS, body, 0)

        for b in range(_NBUF):
            pltpu.make_async_copy(
                table_hbm.at[pl.ds(0, _CHUNK)], bufs.at[b], ssems.at[b]
            ).wait()

    return _sc_gather


def kernel(attrs, attr_emb, W1, b1, W2, b2):
    shift = (jnp.arange(_F, dtype=attrs.dtype) * 1000)[:, None]
    idx = (attrs.T + shift).reshape(_NW, _NCHUNKS, _CHUNK)
    out_table = _mlp_table(attr_emb, W1, b1, W2, b2)
    out_flat = _sc_gather_kernel()(out_table, idx)
    return out_flat.reshape(_F, _B, _D).transpose(1, 0, 2)


# speedup vs baseline: 1.7378x; 1.0497x over previous
import functools
import math

import jax
import jax.numpy as jnp
from jax import lax
from jax.experimental import pallas as pl
from jax.experimental.pallas import tpu as pltpu
from jax.experimental.pallas import tpu_sc as plsc

_B = 4096
_F = 26
_D = 128
_V = 26000
_ROWS = _B * _F

_NC = 2
_NS = 16
_NW = _NC * _NS
_RPW = _ROWS // _NW
_CHUNK = 128
_NCHUNKS = _RPW // _CHUNK
_NBUF = 4

_MLP_ROWS = 5200
_MLP_GRID = _V // _MLP_ROWS

_INV_SQRT2 = 1.0 / math.sqrt(2.0)


def _mlp_body(x_ref, w1_ref, b1_ref, w2_ref, b2_ref, o_ref):
    x = x_ref[...]
    h = jnp.dot(x, w1_ref[...], preferred_element_type=jnp.float32) + b1_ref[...]
    h = 0.5 * h * (1.0 + lax.erf(h * _INV_SQRT2))
    o_ref[...] = jnp.dot(h, w2_ref[...], preferred_element_type=jnp.float32) + b2_ref[...]


def _mlp_table(attr_emb, W1, b1, W2, b2):
    return pl.pallas_call(
        _mlp_body,
        grid=(_MLP_GRID,),
        in_specs=[
            pl.BlockSpec((_MLP_ROWS, _D), lambda i: (i, 0)),
            pl.BlockSpec((_D, _D), lambda i: (0, 0)),
            pl.BlockSpec((1, _D), lambda i: (0, 0)),
            pl.BlockSpec((_D, _D), lambda i: (0, 0)),
            pl.BlockSpec((1, _D), lambda i: (0, 0)),
        ],
        out_specs=pl.BlockSpec((_MLP_ROWS, _D), lambda i: (i, 0)),
        out_shape=jax.ShapeDtypeStruct((_V, _D), jnp.float32),
    )(attr_emb, W1, b1[None, :], W2, b2[None, :])


@functools.lru_cache(maxsize=1)
def _sc_gather_kernel():
    @functools.partial(
        pl.kernel,
        out_type=jax.ShapeDtypeStruct((_ROWS, _D), jnp.float32),
        mesh=plsc.VectorSubcoreMesh(core_axis_name="c", subcore_axis_name="s"),
        scratch_types=[
            pltpu.VMEM((_NCHUNKS, _CHUNK), jnp.int32),
            pltpu.VMEM((_NBUF, _CHUNK, _D), jnp.float32),
            pltpu.SemaphoreType.DMA((_NBUF,)),
            pltpu.SemaphoreType.DMA((_NBUF,)),
        ],
    )
    def _sc_gather(table_hbm, idx_hbm, out_hbm, idx_v, bufs, gsems, ssems):
        wid = lax.axis_index("s") * _NC + lax.axis_index("c")
        base = wid * _RPW
        pltpu.sync_copy(idx_hbm.at[wid], idx_v)

        for b in range(_NBUF):
            pltpu.async_copy(table_hbm.at[idx_v.at[b]], bufs.at[b], gsems.at[b])

        def body(j, carry):
            b = lax.rem(j, _NBUF)
            pltpu.make_async_copy(
                table_hbm.at[pl.ds(0, _CHUNK)], bufs.at[b], gsems.at[b]
            ).wait()
            pltpu.async_copy(
                bufs.at[b], out_hbm.at[pl.ds(base + j * _CHUNK, _CHUNK)], ssems.at[b]
            )

            @pl.when(j + _NBUF < _NCHUNKS)
            def _():
                pltpu.make_async_copy(
                    table_hbm.at[pl.ds(0, _CHUNK)], bufs.at[b], ssems.at[b]
                ).wait()
                pltpu.async_copy(
                    table_hbm.at[idx_v.at[j + _NBUF]], bufs.at[b], gsems.at[b]
                )

            return carry

        lax.fori_loop(0, _NCHUNKS, body, 0)

        for b in range(_NBUF):
            pltpu.make_async_copy(
                table_hbm.at[pl.ds(0, _CHUNK)], bufs.at[b], ssems.at[b]
            ).wait()

    return _sc_gather


def kernel(attrs, attr_emb, W1, b1, W2, b2):
    shift = (jnp.arange(_F, dtype=attrs.dtype) * 1000)[:, None]
    idx = (attrs.T + shift).reshape(_NW, _NCHUNKS, _CHUNK)
    out_table = _mlp_table(attr_emb, W1, b1, W2, b2)
    out_flat = _sc_gather_kernel()(out_table, idx)
    return out_flat.reshape(_F, _B, _D).transpose(1, 0, 2)


# speedup vs baseline: 1.7924x; 1.0314x over previous
import functools
import math

import jax
import jax.numpy as jnp
from jax import lax
from jax.experimental import pallas as pl
from jax.experimental.pallas import tpu as pltpu
from jax.experimental.pallas import tpu_sc as plsc

_B = 4096
_F = 26
_D = 128
_V = 26000
_ROWS = _B * _F

_NC = 2
_NS = 16
_NW = _NC * _NS
_RPW = _ROWS // _NW
_CHUNK = 128
_NCHUNKS = _RPW // _CHUNK
_NBUF = 6

_MLP_ROWS = 5200
_MLP_GRID = _V // _MLP_ROWS

_INV_SQRT2 = 1.0 / math.sqrt(2.0)


def _mlp_body(x_ref, w1_ref, b1_ref, w2_ref, b2_ref, o_ref):
    x = x_ref[...]
    h = jnp.dot(x, w1_ref[...], preferred_element_type=jnp.float32) + b1_ref[...]
    h = 0.5 * h * (1.0 + lax.erf(h * _INV_SQRT2))
    o_ref[...] = jnp.dot(h, w2_ref[...], preferred_element_type=jnp.float32) + b2_ref[...]


def _mlp_table(attr_emb, W1, b1, W2, b2):
    return pl.pallas_call(
        _mlp_body,
        grid=(_MLP_GRID,),
        in_specs=[
            pl.BlockSpec((_MLP_ROWS, _D), lambda i: (i, 0)),
            pl.BlockSpec((_D, _D), lambda i: (0, 0)),
            pl.BlockSpec((1, _D), lambda i: (0, 0)),
            pl.BlockSpec((_D, _D), lambda i: (0, 0)),
            pl.BlockSpec((1, _D), lambda i: (0, 0)),
        ],
        out_specs=pl.BlockSpec((_MLP_ROWS, _D), lambda i: (i, 0)),
        out_shape=jax.ShapeDtypeStruct((_V, _D), jnp.float32),
    )(attr_emb, W1, b1[None, :], W2, b2[None, :])


@functools.lru_cache(maxsize=1)
def _sc_gather_kernel():
    @functools.partial(
        pl.kernel,
        out_type=jax.ShapeDtypeStruct((_ROWS, _D), jnp.float32),
        mesh=plsc.VectorSubcoreMesh(core_axis_name="c", subcore_axis_name="s"),
        scratch_types=[
            pltpu.VMEM((_NCHUNKS, _CHUNK), jnp.int32),
            pltpu.VMEM((_NBUF, _CHUNK, _D), jnp.float32),
            pltpu.SemaphoreType.DMA((_NBUF,)),
            pltpu.SemaphoreType.DMA((_NBUF,)),
        ],
    )
    def _sc_gather(table_hbm, idx_hbm, out_hbm, idx_v, bufs, gsems, ssems):
        wid = lax.axis_index("s") * _NC + lax.axis_index("c")
        base = wid * _RPW
        pltpu.sync_copy(idx_hbm.at[wid], idx_v)

        for b in range(_NBUF):
            pltpu.async_copy(table_hbm.at[idx_v.at[b]], bufs.at[b], gsems.at[b])

        def body(j, carry):
            b = lax.rem(j, _NBUF)
            pltpu.make_async_copy(
                table_hbm.at[pl.ds(0, _CHUNK)], bufs.at[b], gsems.at[b]
            ).wait()
            pltpu.async_copy(
                bufs.at[b], out_hbm.at[pl.ds(base + j * _CHUNK, _CHUNK)], ssems.at[b]
            )

            @pl.when(j + _NBUF < _NCHUNKS)
            def _():
                pltpu.make_async_copy(
                    table_hbm.at[pl.ds(0, _CHUNK)], bufs.at[b], ssems.at[b]
                ).wait()
                pltpu.async_copy(
                    table_hbm.at[idx_v.at[j + _NBUF]], bufs.at[b], gsems.at[b]
                )

            return carry

        lax.fori_loop(0, _NCHUNKS, body, 0)

        for b in range(_NBUF):
            pltpu.make_async_copy(
                table_hbm.at[pl.ds(0, _CHUNK)], bufs.at[b], ssems.at[b]
            ).wait()

    return _sc_gather


def kernel(attrs, attr_emb, W1, b1, W2, b2):
    shift = (jnp.arange(_F, dtype=attrs.dtype) * 1000)[:, None]
    idx = (attrs.T + shift).reshape(_NW, _NCHUNKS, _CHUNK)
    out_table = _mlp_table(attr_emb, W1, b1, W2, b2)
    out_flat = _sc_gather_kernel()(out_table, idx)
    return out_flat.reshape(_F, _B, _D).transpose(1, 0, 2)
